# Initial kernel scaffold; baseline (speedup 1.0000x reference)
import jax
import jax.numpy as jnp
from jax.experimental import pallas as pl


def kernel(x, edge_index, edge_attr, batch, gn_weight, gn_bias, gn_mean_scale, bn_weight, bn_bias, W_pass, b_pass, t, W1, ln_w, ln_b, W2):
    raise NotImplementedError("write your pallas kernel here")


# speedup vs baseline: 3.7017x; 3.7017x over previous
import functools

import jax
import jax.numpy as jnp
from jax import lax
from jax.experimental import pallas as pl
from jax.experimental.pallas import tpu as pltpu
from jax.experimental.pallas import tpu_sc as plsc

f32 = jnp.float32

N = 10000
E = 320000
D = 128
G = 16

NS = 16
EPT = E // NS
C = 80
NCHUNK = EPT // C

EB = 2000
NBE = E // EB
NB = 2000
NBN = N // NB


def _gn_body(x_ref, brow_ref, bcol_ref, gw_ref, gb_ref, gms_ref, h_ref):
    x = x_ref[...]
    brow = brow_ref[...]
    bcol = bcol_ref[...]
    oh_gn = (lax.broadcasted_iota(jnp.int32, (G, N), 0) == brow).astype(f32)
    oh_ng = (lax.broadcasted_iota(jnp.int32, (N, G), 1) == bcol).astype(f32)
    cnt = jnp.maximum(jnp.sum(oh_gn, axis=1, keepdims=True), 1.0)
    dn = (((1,), (0,)), ((), ()))
    mean = lax.dot_general(oh_gn, x, dn, preferred_element_type=f32) / cnt
    meanb = lax.dot_general(oh_ng, mean, dn, preferred_element_type=f32)
    outn = x - meanb * gms_ref[...]
    var = lax.dot_general(oh_gn, outn * outn, dn,
                          preferred_element_type=f32) / cnt
    rstd = lax.rsqrt(var + 1e-5)
    rstdb = lax.dot_general(oh_ng, rstd, dn, preferred_element_type=f32)
    h = gw_ref[...] * outn * rstdb + gb_ref[...]
    h_ref[...] = jnp.where(h >= 0, h, 0.01 * h)


def _graphnorm(x, batch, gn_weight, gn_bias, gn_mean_scale):
    return pl.pallas_call(
        _gn_body,
        out_shape=jax.ShapeDtypeStruct((N, D), f32),
    )(x, batch.reshape(1, N), batch.reshape(N, 1),
      gn_weight.reshape(1, D), gn_bias.reshape(1, D),
      gn_mean_scale.reshape(1, D))


def _bn_stats_body(ea_ref, bw_ref, bb_ref, a_ref, c_ref, acc_ref):
    i = pl.program_id(0)
    blk = ea_ref[...]

    @pl.when(i == 0)
    def _():
        acc_ref[...] = jnp.zeros_like(acc_ref)

    s1 = jnp.sum(blk, axis=0, keepdims=True)
    s2 = jnp.sum(blk * blk, axis=0, keepdims=True)
    acc_ref[0:1, :] = acc_ref[0:1, :] + s1
    acc_ref[1:2, :] = acc_ref[1:2, :] + s2

    @pl.when(i == NBE - 1)
    def _():
        mu = acc_ref[0:1, :] / E
        var = acc_ref[1:2, :] / E - mu * mu
        a = bw_ref[...] * lax.rsqrt(var + 1e-5)
        a_ref[...] = a
        c_ref[...] = bb_ref[...] - mu * a


def _bn_stats(edge_attr, bn_weight, bn_bias):
    return pl.pallas_call(
        _bn_stats_body,
        grid=(NBE,),
        in_specs=[
            pl.BlockSpec((EB, D), lambda i: (i, 0)),
            pl.BlockSpec((1, D), lambda i: (0, 0)),
            pl.BlockSpec((1, D), lambda i: (0, 0)),
        ],
        out_specs=[
            pl.BlockSpec((1, D), lambda i: (0, 0)),
            pl.BlockSpec((1, D), lambda i: (0, 0)),
        ],
        out_shape=[
            jax.ShapeDtypeStruct((1, D), f32),
            jax.ShapeDtypeStruct((1, D), f32),
        ],
        scratch_shapes=[pltpu.VMEM((2, D), f32)],
    )(edge_attr, bn_weight.reshape(1, D), bn_bias.reshape(1, D))


def _attr_body(ea_ref, a_ref, c_ref, wp_ref, bp_ref, attr_ref, oa_ref):
    ea = ea_ref[...]
    y = ea * a_ref[...] + c_ref[...]
    y = jnp.where(y >= 0, y, 0.01 * y)
    attr = lax.dot_general(y, wp_ref[...], (((1,), (1,)), ((), ())),
                           preferred_element_type=f32) + bp_ref[...]
    attr_ref[...] = attr
    oa_ref[...] = attr + ea


def _edge_transform(edge_attr, a, c, W_pass, b_pass):
    return pl.pallas_call(
        _attr_body,
        grid=(NBE,),
        in_specs=[
            pl.BlockSpec((EB, D), lambda i: (i, 0)),
            pl.BlockSpec((1, D), lambda i: (0, 0)),
            pl.BlockSpec((1, D), lambda i: (0, 0)),
            pl.BlockSpec((D, D), lambda i: (0, 0)),
            pl.BlockSpec((1, D), lambda i: (0, 0)),
        ],
        out_specs=[
            pl.BlockSpec((EB, D), lambda i: (i, 0)),
            pl.BlockSpec((EB, D), lambda i: (i, 0)),
        ],
        out_shape=[
            jax.ShapeDtypeStruct((E, D), f32),
            jax.ShapeDtypeStruct((E, D), f32),
        ],
    )(edge_attr, a, c, W_pass, b_pass.reshape(1, D))


def _sc_edge_body(src_hbm, dst_hbm, h_hbm, attr_hbm, t_hbm, z_hbm,
                  s_out, e_out,
                  src_v, dst_v, h_v, a_v, o_v, t_v, acc_sh, sem):
    cid = lax.axis_index("c")
    sid = lax.axis_index("s")

    @pl.when(sid == 0)
    def _():
        pltpu.sync_copy(z_hbm, acc_sh)

    pltpu.sync_copy(t_hbm, t_v)
    plsc.subcore_barrier()

    def _process(store_pm, out_hbm):
        def chunk_body(k, carry):
            base = sid * EPT + k * C
            pltpu.sync_copy(src_hbm.at[pl.ds(base, C)], src_v)
            pltpu.sync_copy(dst_hbm.at[pl.ds(base, C)], dst_v)
            pltpu.async_copy(h_hbm.at[src_v], h_v, sem).wait()
            pltpu.sync_copy(attr_hbm.at[pl.ds(base, C)], a_v)
            tv = t_v[...]

            def edge_body(i, carry2):
                for j in range(D // 16):
                    hs = h_v[i, pl.ds(j * 16, 16)]
                    at = a_v[i, pl.ds(j * 16, 16)]
                    msg = jnp.maximum(hs + at, 0.0) + 1e-7
                    p = jnp.exp(msg * tv)
                    o_v[i, pl.ds(j * 16, 16)] = p * msg if store_pm else p
                return carry2

            lax.fori_loop(0, C, edge_body, 0)
            pltpu.sync_copy(o_v, acc_sh.at[dst_v], add=True)
            return carry

        lax.fori_loop(0, NCHUNK, chunk_body, 0)
        plsc.subcore_barrier()

        @pl.when(sid == 0)
        def _():
            pltpu.sync_copy(acc_sh, out_hbm)

    @pl.when(cid == 0)
    def _():
        _process(False, s_out)

    @pl.when(cid == 1)
    def _():
        _process(True, e_out)


def _sc_edge_pass(src, dst, h, attr, tvec, zrows):
    mesh = plsc.VectorSubcoreMesh(core_axis_name="c", subcore_axis_name="s",
                                  num_cores=2, num_subcores=NS)
    k = functools.partial(
        pl.kernel,
        out_type=[jax.ShapeDtypeStruct((N, D), f32)] * 2,
        mesh=mesh,
        scratch_types=[
            pltpu.VMEM((C,), jnp.int32),
            pltpu.VMEM((C,), jnp.int32),
            pltpu.VMEM((C, D), f32),
            pltpu.VMEM((C, D), f32),
            pltpu.VMEM((C, D), f32),
            pltpu.VMEM((16,), f32),
            pltpu.VMEM_SHARED((N, D), f32),
            pltpu.SemaphoreType.DMA,
        ],
    )(_sc_edge_body)
    return k(src, dst, h, attr, tvec, zrows)


def _final_body(s_ref, e_ref, h_ref, x_ref, w1_ref, lnw_ref, lnb_ref, w2_ref,
                o_ref):
    out2 = e_ref[...] / (s_ref[...] + 1e-16) + h_ref[...]
    h1 = lax.dot_general(out2, w1_ref[...], (((1,), (1,)), ((), ())),
                         preferred_element_type=f32)
    mu = jnp.mean(h1, axis=1, keepdims=True)
    dlt = h1 - mu
    v = jnp.mean(dlt * dlt, axis=1, keepdims=True)
    h1n = dlt * lax.rsqrt(v + 1e-5) * lnw_ref[...] + lnb_ref[...]
    h1n = jnp.maximum(h1n, 0.0)
    h2 = lax.dot_general(h1n, w2_ref[...], (((1,), (1,)), ((), ())),
                         preferred_element_type=f32)
    o_ref[...] = h2 + x_ref[...]


def _final_stage(s, e, h, x, W1, ln_w, ln_b, W2):
    return pl.pallas_call(
        _final_body,
        grid=(NBN,),
        in_specs=[
            pl.BlockSpec((NB, D), lambda i: (i, 0)),
            pl.BlockSpec((NB, D), lambda i: (i, 0)),
            pl.BlockSpec((NB, D), lambda i: (i, 0)),
            pl.BlockSpec((NB, D), lambda i: (i, 0)),
            pl.BlockSpec((2 * D, D), lambda i: (0, 0)),
            pl.BlockSpec((1, 2 * D), lambda i: (0, 0)),
            pl.BlockSpec((1, 2 * D), lambda i: (0, 0)),
            pl.BlockSpec((D, 2 * D), lambda i: (0, 0)),
        ],
        out_specs=pl.BlockSpec((NB, D), lambda i: (i, 0)),
        out_shape=jax.ShapeDtypeStruct((N, D), f32),
    )(s, e, h, x, W1, ln_w.reshape(1, 2 * D), ln_b.reshape(1, 2 * D), W2)


def kernel(x, edge_index, edge_attr, batch, gn_weight, gn_bias, gn_mean_scale,
           bn_weight, bn_bias, W_pass, b_pass, t, W1, ln_w, ln_b, W2):
    src = edge_index[0]
    dst = edge_index[1]
    h = _graphnorm(x, batch, gn_weight, gn_bias, gn_mean_scale)
    a, c = _bn_stats(edge_attr, bn_weight, bn_bias)
    attr, out_attr = _edge_transform(edge_attr, a, c, W_pass, b_pass)
    tvec = jnp.broadcast_to(t.astype(f32), (16,))
    zrows = jnp.zeros((N, D), f32)
    s, e = _sc_edge_pass(src, dst, h, attr, tvec, zrows)
    out1 = _final_stage(s, e, h, x, W1, ln_w, ln_b, W2)
    return (out1, out_attr)
